# Initial kernel scaffold; baseline (speedup 1.0000x reference)
import functools

import jax
import jax.numpy as jnp
from jax import lax
from jax.experimental import pallas as pl
from jax.experimental.pallas import tpu as pltpu
from jax.experimental.pallas import tpu_sc as plsc

N = 10000
E = 320000
D = 128
NCLS = 16
NET = 8
NPAD = 10240
NCORES = 2
NSUB = 16
CHUNK = 128
NCHUNKS = E // CHUNK
CPC = NCHUNKS // NCORES
ITERS = (CPC + NSUB - 1) // NSUB
ROWS_PER_TILE = NPAD // NSUB
DEGW = 16
ZROWS = 64

RB = 512
GRID = NPAD // RB


def _mesh():
    return plsc.VectorSubcoreMesh(core_axis_name="c", subcore_axis_name="s")


def _sc_prep(src, dst, e_feat):
    @functools.partial(
        pl.kernel,
        out_type=(
            jax.ShapeDtypeStruct((NCORES, NPAD, DEGW), jnp.float32),
            jax.ShapeDtypeStruct((NCORES, NPAD, DEGW), jnp.float32),
            jax.ShapeDtypeStruct((E,), jnp.int32),
        ),
        mesh=_mesh(),
        scratch_types=[
            pltpu.VMEM((1, CHUNK), jnp.int32),
            pltpu.VMEM((1, CHUNK), jnp.int32),
            pltpu.VMEM((CHUNK,), jnp.int32),
            pltpu.VMEM((CHUNK,), jnp.int32),
            pltpu.VMEM((CHUNK, DEGW), jnp.float32),
            pltpu.VMEM((ZROWS, DEGW), jnp.float32),
            pltpu.VMEM_SHARED((NPAD, DEGW), jnp.float32),
            pltpu.VMEM_SHARED((NPAD, DEGW), jnp.float32),
        ],
    )
    def prep(src_h, dst_h, ef_h, dego_h, degi_h, gidx_h,
             src_v, dst_v, ef_v, gx_v, ones_v, zz_v, dego_sh, degi_sh):
        c = lax.axis_index("c")
        s = lax.axis_index("s")

        one16 = jnp.ones((DEGW,), jnp.float32)
        zero16 = jnp.zeros((DEGW,), jnp.float32)

        def fill_ones(i, _):
            ones_v[i, :] = one16
            return 0
        lax.fori_loop(0, CHUNK, fill_ones, 0)

        def fill_zeros(i, _):
            zz_v[i, :] = zero16
            return 0
        lax.fori_loop(0, ZROWS, fill_zeros, 0)

        def zslice(k, _):
            base = s * ROWS_PER_TILE + k * ZROWS
            pltpu.sync_copy(zz_v, dego_sh.at[pl.ds(base, ZROWS)])
            pltpu.sync_copy(zz_v, degi_sh.at[pl.ds(base, ZROWS)])
            return 0
        lax.fori_loop(0, ROWS_PER_TILE // ZROWS, zslice, 0)
        plsc.subcore_barrier()

        def body(it, _):
            cid = it * NSUB + s

            @pl.when(cid < CPC)
            def _():
                ebase = (c * CPC + cid) * CHUNK
                pltpu.sync_copy(src_h.at[pl.ds(ebase, CHUNK)], src_v.at[0])
                pltpu.sync_copy(dst_h.at[pl.ds(ebase, CHUNK)], dst_v.at[0])
                pltpu.sync_copy(ef_h.at[pl.ds(ebase, CHUNK)], ef_v)

                def g(j, _):
                    sl = pl.ds(j * 16, 16)
                    gx_v[sl] = ef_v[sl] * NPAD + src_v[0, sl]
                    return 0
                lax.fori_loop(0, CHUNK // 16, g, 0)
                pltpu.sync_copy(gx_v, gidx_h.at[pl.ds(ebase, CHUNK)])

                pltpu.sync_copy(ones_v, dego_sh.at[src_v.at[0]], add=True)
                pltpu.sync_copy(ones_v, degi_sh.at[dst_v.at[0]], add=True)
            return 0
        lax.fori_loop(0, ITERS, body, 0)

        plsc.subcore_barrier()
        rbase = s * ROWS_PER_TILE
        pltpu.sync_copy(dego_sh.at[pl.ds(rbase, ROWS_PER_TILE)],
                        dego_h.at[c].at[pl.ds(rbase, ROWS_PER_TILE)])
        pltpu.sync_copy(degi_sh.at[pl.ds(rbase, ROWS_PER_TILE)],
                        degi_h.at[c].at[pl.ds(rbase, ROWS_PER_TILE)])

    return prep(src, dst, e_feat)


def _sc_conv(tbl, gidx, dst):
    @functools.partial(
        pl.kernel,
        out_type=jax.ShapeDtypeStruct((NCORES, NPAD, D), jnp.float32),
        mesh=_mesh(),
        scratch_types=[
            pltpu.VMEM((1, CHUNK), jnp.int32),
            pltpu.VMEM((1, CHUNK), jnp.int32),
            pltpu.VMEM((CHUNK, D), jnp.float32),
            pltpu.VMEM((ZROWS, D), jnp.float32),
            pltpu.VMEM_SHARED((NPAD, D), jnp.float32),
            pltpu.SemaphoreType.DMA,
        ],
    )
    def conv(tbl_h, gidx_h, dst_h, out_h, gx_v, dst_v, rows_v, zz_v, agg_sh, sem):
        c = lax.axis_index("c")
        s = lax.axis_index("s")

        zero16 = jnp.zeros((16,), jnp.float32)

        def fill_zeros(k, _):
            zz_v[k // (D // 16), pl.ds((k % (D // 16)) * 16, 16)] = zero16
            return 0
        lax.fori_loop(0, ZROWS * (D // 16), fill_zeros, 0)

        def zslice(k, _):
            base = s * ROWS_PER_TILE + k * ZROWS
            pltpu.sync_copy(zz_v, agg_sh.at[pl.ds(base, ZROWS)])
            return 0
        lax.fori_loop(0, ROWS_PER_TILE // ZROWS, zslice, 0)
        plsc.subcore_barrier()

        def body(it, _):
            cid = it * NSUB + s

            @pl.when(cid < CPC)
            def _():
                ebase = (c * CPC + cid) * CHUNK
                pltpu.sync_copy(gidx_h.at[pl.ds(ebase, CHUNK)], gx_v.at[0])
                pltpu.sync_copy(dst_h.at[pl.ds(ebase, CHUNK)], dst_v.at[0])
                pltpu.async_copy(tbl_h.at[gx_v.at[0]], rows_v, sem).wait()
                pltpu.sync_copy(rows_v, agg_sh.at[dst_v.at[0]], add=True)
            return 0
        lax.fori_loop(0, ITERS, body, 0)

        plsc.subcore_barrier()
        rbase = s * ROWS_PER_TILE
        pltpu.sync_copy(agg_sh.at[pl.ds(rbase, ROWS_PER_TILE)],
                        out_h.at[c].at[pl.ds(rbase, ROWS_PER_TILE)])

    return conv(tbl, gidx, dst)


def _tc_norms(dego, degi):
    def body(do_ref, di_ref, ns_ref, nd_ref):
        d_o = do_ref[0] + do_ref[1]
        d_i = di_ref[0] + di_ref[1]
        n_o = lax.rsqrt(jnp.where(d_o > 0, d_o, 1.0))
        n_i = lax.rsqrt(jnp.where(d_i > 0, d_i, 1.0))
        ns_ref[...] = jnp.concatenate([n_o] * (D // DEGW), axis=1)
        nd_ref[...] = jnp.concatenate([n_i] * (D // DEGW), axis=1)

    return pl.pallas_call(
        body,
        grid=(GRID,),
        in_specs=[pl.BlockSpec((NCORES, RB, DEGW), lambda i: (0, i, 0)),
                  pl.BlockSpec((NCORES, RB, DEGW), lambda i: (0, i, 0))],
        out_specs=[pl.BlockSpec((RB, D), lambda i: (i, 0)),
                   pl.BlockSpec((RB, D), lambda i: (i, 0))],
        out_shape=[jax.ShapeDtypeStruct((NPAD, D), jnp.float32),
                   jax.ShapeDtypeStruct((NPAD, D), jnp.float32)],
    )(dego, degi)


def _tc_table0(x0p, wT, b, ns, et):
    def body(x_ref, w_ref, b_ref, ns_ref, et_ref, out_ref):
        h = jnp.dot(x_ref[...], w_ref[...], preferred_element_type=jnp.float32)
        hs = (h + b_ref[...]) * ns_ref[...]
        for t in range(NET):
            out_ref[t] = hs * et_ref[t]

    return pl.pallas_call(
        body,
        grid=(GRID,),
        in_specs=[pl.BlockSpec((RB, D), lambda i: (i, 0)),
                  pl.BlockSpec((D, D), lambda i: (0, 0)),
                  pl.BlockSpec((1, D), lambda i: (0, 0)),
                  pl.BlockSpec((RB, D), lambda i: (i, 0)),
                  pl.BlockSpec(memory_space=pltpu.SMEM)],
        out_specs=pl.BlockSpec((NET, RB, D), lambda i: (0, i, 0)),
        out_shape=jax.ShapeDtypeStruct((NET, NPAD, D), jnp.float32),
    )(x0p, wT, b.reshape(1, D), ns, et)


def _tc_table_l1(agg, nd, ns, et):
    def body(a_ref, nd_ref, ns_ref, et_ref, out_ref):
        h = (a_ref[0] + a_ref[1]) * nd_ref[...]
        hs = h * ns_ref[...]
        for t in range(NET):
            out_ref[t] = hs * et_ref[t]

    return pl.pallas_call(
        body,
        grid=(GRID,),
        in_specs=[pl.BlockSpec((NCORES, RB, D), lambda i: (0, i, 0)),
                  pl.BlockSpec((RB, D), lambda i: (i, 0)),
                  pl.BlockSpec((RB, D), lambda i: (i, 0)),
                  pl.BlockSpec(memory_space=pltpu.SMEM)],
        out_specs=pl.BlockSpec((NET, RB, D), lambda i: (0, i, 0)),
        out_shape=jax.ShapeDtypeStruct((NET, NPAD, D), jnp.float32),
    )(agg, nd, ns, et)


def _tc_table_l2(agg, nd, w1, b1, ns, et):
    def body(a_ref, nd_ref, w_ref, b_ref, ns_ref, et_ref, out_ref):
        hin = (a_ref[0] + a_ref[1]) * nd_ref[...]
        h = jnp.dot(hin, w_ref[...], preferred_element_type=jnp.float32) + b_ref[...]
        hs = jnp.maximum(h, 0.0) * ns_ref[...]
        for t in range(NET):
            out_ref[t] = hs * et_ref[t]

    return pl.pallas_call(
        body,
        grid=(GRID,),
        in_specs=[pl.BlockSpec((NCORES, RB, D), lambda i: (0, i, 0)),
                  pl.BlockSpec((RB, D), lambda i: (i, 0)),
                  pl.BlockSpec((D, D), lambda i: (0, 0)),
                  pl.BlockSpec((1, D), lambda i: (0, 0)),
                  pl.BlockSpec((RB, D), lambda i: (i, 0)),
                  pl.BlockSpec(memory_space=pltpu.SMEM)],
        out_specs=pl.BlockSpec((NET, RB, D), lambda i: (0, i, 0)),
        out_shape=jax.ShapeDtypeStruct((NET, NPAD, D), jnp.float32),
    )(agg, nd, w1, b1.reshape(1, D), ns, et)


def _tc_final(agg, nd, w2p, b2p):
    def body(a_ref, nd_ref, w_ref, b_ref, out_ref):
        hin = (a_ref[0] + a_ref[1]) * nd_ref[...]
        out_ref[...] = jnp.dot(hin, w_ref[...],
                               preferred_element_type=jnp.float32) + b_ref[...]

    return pl.pallas_call(
        body,
        grid=(GRID,),
        in_specs=[pl.BlockSpec((NCORES, RB, D), lambda i: (0, i, 0)),
                  pl.BlockSpec((RB, D), lambda i: (i, 0)),
                  pl.BlockSpec((D, D), lambda i: (0, 0)),
                  pl.BlockSpec((1, D), lambda i: (0, 0))],
        out_specs=pl.BlockSpec((RB, D), lambda i: (i, 0)),
        out_shape=jax.ShapeDtypeStruct((NPAD, D), jnp.float32),
    )(agg, nd, w2p, b2p.reshape(1, D))


def kernel(x0, edge_index, e_feat, W_fc0, b_fc0, et0, et1, et2, W1, b1, W2, b2):
    src = edge_index[0]
    dst = edge_index[1]
    x0p = jnp.pad(x0, ((0, NPAD - N), (0, 0)))
    w2p = jnp.pad(W2, ((0, 0), (0, D - NCLS)))
    b2p = jnp.pad(b2, ((0, D - NCLS),))

    dego, degi, gidx = _sc_prep(src, dst, e_feat)
    ns, nd = _tc_norms(dego, degi)

    tbl0 = _tc_table0(x0p, W_fc0.T, b_fc0, ns, et0).reshape(NET * NPAD, D)
    agg0 = _sc_conv(tbl0, gidx, dst)
    tbl1 = _tc_table_l1(agg0, nd, ns, et1).reshape(NET * NPAD, D)
    agg1 = _sc_conv(tbl1, gidx, dst)
    tbl2 = _tc_table_l2(agg1, nd, ns, et2, w1=W1, b1=b1) if False else _tc_table_l2(agg1, nd, W1, b1, ns, et2)
    tbl2 = tbl2.reshape(NET * NPAD, D)
    agg2 = _sc_conv(tbl2, gidx, dst)
    out = _tc_final(agg2, nd, w2p, b2p)
    return out[:N, :NCLS]


# speedup vs baseline: 6.6968x; 6.6968x over previous
import functools

import jax
import jax.numpy as jnp
from jax import lax
from jax.experimental import pallas as pl
from jax.experimental.pallas import tpu as pltpu
from jax.experimental.pallas import tpu_sc as plsc

N = 10000
E = 320000
D = 128
NCLS = 16
NET = 8
NPAD = 10240
NCORES = 2
NSUB = 16
CHUNK = 128
NCHUNKS = E // CHUNK
CPC = NCHUNKS // NCORES
ITERS = (CPC + NSUB - 1) // NSUB
PREP_ITERS = (NCHUNKS + NSUB - 1) // NSUB
ROWS_PER_TILE = NPAD // NSUB
DEGW = 16
ZROWS = 64

RB = 512
GRID = NPAD // RB


def _mesh():
    return plsc.VectorSubcoreMesh(core_axis_name="c", subcore_axis_name="s")


def _sc_prep(src, dst, e_feat, ones_in, zeros_in):
    @functools.partial(
        pl.kernel,
        out_type=(
            jax.ShapeDtypeStruct((NCORES, NPAD, D), jnp.float32),
            jax.ShapeDtypeStruct((E,), jnp.int32),
        ),
        mesh=_mesh(),
        scratch_types=[
            pltpu.VMEM((1, CHUNK), jnp.int32),
            pltpu.VMEM((CHUNK,), jnp.int32),
            pltpu.VMEM((CHUNK,), jnp.int32),
            pltpu.VMEM((CHUNK, D), jnp.float32),
            pltpu.VMEM((ZROWS, D), jnp.float32),
            pltpu.VMEM_SHARED((NPAD, D), jnp.float32),
        ],
    )
    def prep(src_h, dst_h, ef_h, ones_h, zeros_h, deg_h, gidx_h,
             idx_v, ef_v, gx_v, ones_v, zz_v, deg_sh):
        c = lax.axis_index("c")
        s = lax.axis_index("s")

        pltpu.sync_copy(ones_h, ones_v)
        pltpu.sync_copy(zeros_h, zz_v)

        def zslice(k, _):
            base = s * ROWS_PER_TILE + k * ZROWS
            pltpu.sync_copy(zz_v, deg_sh.at[pl.ds(base, ZROWS)])
            return 0
        lax.fori_loop(0, ROWS_PER_TILE // ZROWS, zslice, 0)
        plsc.subcore_barrier()

        def body(it, _):
            cid = it * NSUB + s

            @pl.when(cid < NCHUNKS)
            def _():
                ebase = cid * CHUNK

                @pl.when(c == 0)
                def _():
                    pltpu.sync_copy(src_h.at[pl.ds(ebase, CHUNK)], idx_v.at[0])
                    pltpu.sync_copy(ef_h.at[pl.ds(ebase, CHUNK)], ef_v)

                    def g(j, _):
                        sl = pl.ds(j * 16, 16)
                        gx_v[sl] = ef_v[sl] * NPAD + idx_v[0, sl]
                        return 0
                    lax.fori_loop(0, CHUNK // 16, g, 0)
                    pltpu.sync_copy(gx_v, gidx_h.at[pl.ds(ebase, CHUNK)])

                @pl.when(c == 1)
                def _():
                    pltpu.sync_copy(dst_h.at[pl.ds(ebase, CHUNK)], idx_v.at[0])

                pltpu.sync_copy(ones_v, deg_sh.at[idx_v.at[0]], add=True)
            return 0
        lax.fori_loop(0, PREP_ITERS, body, 0)

        plsc.subcore_barrier()
        rbase = s * ROWS_PER_TILE
        pltpu.sync_copy(deg_sh.at[pl.ds(rbase, ROWS_PER_TILE)],
                        deg_h.at[c].at[pl.ds(rbase, ROWS_PER_TILE)])

    return prep(src, dst, e_feat, ones_in, zeros_in)


def _sc_conv(tbl, gidx, dst):
    @functools.partial(
        pl.kernel,
        out_type=jax.ShapeDtypeStruct((NCORES, NPAD, D), jnp.float32),
        mesh=_mesh(),
        scratch_types=[
            pltpu.VMEM((1, CHUNK), jnp.int32),
            pltpu.VMEM((1, CHUNK), jnp.int32),
            pltpu.VMEM((CHUNK, D), jnp.float32),
            pltpu.VMEM((ZROWS, D), jnp.float32),
            pltpu.VMEM_SHARED((NPAD, D), jnp.float32),
            pltpu.SemaphoreType.DMA,
        ],
    )
    def conv(tbl_h, gidx_h, dst_h, out_h, gx_v, dst_v, rows_v, zz_v, agg_sh, sem):
        c = lax.axis_index("c")
        s = lax.axis_index("s")

        zero16 = jnp.zeros((16,), jnp.float32)

        def fill_zeros(k, _):
            zz_v[k // (D // 16), pl.ds((k % (D // 16)) * 16, 16)] = zero16
            return 0
        lax.fori_loop(0, ZROWS * (D // 16), fill_zeros, 0)

        def zslice(k, _):
            base = s * ROWS_PER_TILE + k * ZROWS
            pltpu.sync_copy(zz_v, agg_sh.at[pl.ds(base, ZROWS)])
            return 0
        lax.fori_loop(0, ROWS_PER_TILE // ZROWS, zslice, 0)
        plsc.subcore_barrier()

        def body(it, _):
            cid = it * NSUB + s

            @pl.when(cid < CPC)
            def _():
                ebase = (c * CPC + cid) * CHUNK
                pltpu.sync_copy(gidx_h.at[pl.ds(ebase, CHUNK)], gx_v.at[0])
                pltpu.sync_copy(dst_h.at[pl.ds(ebase, CHUNK)], dst_v.at[0])
                pltpu.async_copy(tbl_h.at[gx_v.at[0]], rows_v, sem).wait()
                pltpu.sync_copy(rows_v, agg_sh.at[dst_v.at[0]], add=True)
            return 0
        lax.fori_loop(0, ITERS, body, 0)

        plsc.subcore_barrier()
        rbase = s * ROWS_PER_TILE
        pltpu.sync_copy(agg_sh.at[pl.ds(rbase, ROWS_PER_TILE)],
                        out_h.at[c].at[pl.ds(rbase, ROWS_PER_TILE)])

    return conv(tbl, gidx, dst)


def _tc_norms(deg):
    def body(dg_ref, ns_ref, nd_ref):
        d_o = dg_ref[0]
        d_i = dg_ref[1]
        ns_ref[...] = lax.rsqrt(jnp.where(d_o > 0, d_o, 1.0))
        nd_ref[...] = lax.rsqrt(jnp.where(d_i > 0, d_i, 1.0))

    return pl.pallas_call(
        body,
        grid=(GRID,),
        in_specs=[pl.BlockSpec((NCORES, RB, D), lambda i: (0, i, 0))],
        out_specs=[pl.BlockSpec((RB, D), lambda i: (i, 0)),
                   pl.BlockSpec((RB, D), lambda i: (i, 0))],
        out_shape=[jax.ShapeDtypeStruct((NPAD, D), jnp.float32),
                   jax.ShapeDtypeStruct((NPAD, D), jnp.float32)],
    )(deg)


def _tc_table0(x0p, wT, b, ns, et):
    def body(x_ref, w_ref, b_ref, ns_ref, et_ref, out_ref):
        h = jnp.dot(x_ref[...], w_ref[...], preferred_element_type=jnp.float32)
        hs = (h + b_ref[...]) * ns_ref[...]
        for t in range(NET):
            out_ref[t] = hs * et_ref[t]

    return pl.pallas_call(
        body,
        grid=(GRID,),
        in_specs=[pl.BlockSpec((RB, D), lambda i: (i, 0)),
                  pl.BlockSpec((D, D), lambda i: (0, 0)),
                  pl.BlockSpec((1, D), lambda i: (0, 0)),
                  pl.BlockSpec((RB, D), lambda i: (i, 0)),
                  pl.BlockSpec(memory_space=pltpu.SMEM)],
        out_specs=pl.BlockSpec((NET, RB, D), lambda i: (0, i, 0)),
        out_shape=jax.ShapeDtypeStruct((NET, NPAD, D), jnp.float32),
    )(x0p, wT, b.reshape(1, D), ns, et)


def _tc_table_l1(agg, nd, ns, et):
    def body(a_ref, nd_ref, ns_ref, et_ref, out_ref):
        h = (a_ref[0] + a_ref[1]) * nd_ref[...]
        hs = h * ns_ref[...]
        for t in range(NET):
            out_ref[t] = hs * et_ref[t]

    return pl.pallas_call(
        body,
        grid=(GRID,),
        in_specs=[pl.BlockSpec((NCORES, RB, D), lambda i: (0, i, 0)),
                  pl.BlockSpec((RB, D), lambda i: (i, 0)),
                  pl.BlockSpec((RB, D), lambda i: (i, 0)),
                  pl.BlockSpec(memory_space=pltpu.SMEM)],
        out_specs=pl.BlockSpec((NET, RB, D), lambda i: (0, i, 0)),
        out_shape=jax.ShapeDtypeStruct((NET, NPAD, D), jnp.float32),
    )(agg, nd, ns, et)


def _tc_table_l2(agg, nd, w1, b1, ns, et):
    def body(a_ref, nd_ref, w_ref, b_ref, ns_ref, et_ref, out_ref):
        hin = (a_ref[0] + a_ref[1]) * nd_ref[...]
        h = jnp.dot(hin, w_ref[...], preferred_element_type=jnp.float32) + b_ref[...]
        hs = jnp.maximum(h, 0.0) * ns_ref[...]
        for t in range(NET):
            out_ref[t] = hs * et_ref[t]

    return pl.pallas_call(
        body,
        grid=(GRID,),
        in_specs=[pl.BlockSpec((NCORES, RB, D), lambda i: (0, i, 0)),
                  pl.BlockSpec((RB, D), lambda i: (i, 0)),
                  pl.BlockSpec((D, D), lambda i: (0, 0)),
                  pl.BlockSpec((1, D), lambda i: (0, 0)),
                  pl.BlockSpec((RB, D), lambda i: (i, 0)),
                  pl.BlockSpec(memory_space=pltpu.SMEM)],
        out_specs=pl.BlockSpec((NET, RB, D), lambda i: (0, i, 0)),
        out_shape=jax.ShapeDtypeStruct((NET, NPAD, D), jnp.float32),
    )(agg, nd, w1, b1.reshape(1, D), ns, et)


def _tc_final(agg, nd, w2p, b2p):
    def body(a_ref, nd_ref, w_ref, b_ref, out_ref):
        hin = (a_ref[0] + a_ref[1]) * nd_ref[...]
        out_ref[...] = jnp.dot(hin, w_ref[...],
                               preferred_element_type=jnp.float32) + b_ref[...]

    return pl.pallas_call(
        body,
        grid=(GRID,),
        in_specs=[pl.BlockSpec((NCORES, RB, D), lambda i: (0, i, 0)),
                  pl.BlockSpec((RB, D), lambda i: (i, 0)),
                  pl.BlockSpec((D, D), lambda i: (0, 0)),
                  pl.BlockSpec((1, D), lambda i: (0, 0))],
        out_specs=pl.BlockSpec((RB, D), lambda i: (i, 0)),
        out_shape=jax.ShapeDtypeStruct((NPAD, D), jnp.float32),
    )(agg, nd, w2p, b2p.reshape(1, D))


def kernel(x0, edge_index, e_feat, W_fc0, b_fc0, et0, et1, et2, W1, b1, W2, b2):
    src = edge_index[0]
    dst = edge_index[1]
    x0p = jnp.pad(x0, ((0, NPAD - N), (0, 0)))
    w2p = jnp.pad(W2, ((0, 0), (0, D - NCLS)))
    b2p = jnp.pad(b2, ((0, D - NCLS),))

    ones_in = jnp.ones((CHUNK, D), jnp.float32)
    zeros_in = jnp.zeros((ZROWS, D), jnp.float32)
    deg, gidx = _sc_prep(src, dst, e_feat, ones_in, zeros_in)
    ns, nd = _tc_norms(deg)

    tbl0 = _tc_table0(x0p, W_fc0.T, b_fc0, ns, et0).reshape(NET * NPAD, D)
    agg0 = _sc_conv(tbl0, gidx, dst)
    tbl1 = _tc_table_l1(agg0, nd, ns, et1).reshape(NET * NPAD, D)
    agg1 = _sc_conv(tbl1, gidx, dst)
    tbl2 = _tc_table_l2(agg1, nd, W1, b1, ns, et2).reshape(NET * NPAD, D)
    agg2 = _sc_conv(tbl2, gidx, dst)
    out = _tc_final(agg2, nd, w2p, b2p)
    return out[:N, :NCLS]
